# baseline (speedup 1.0000x reference)
import functools

import jax
import jax.numpy as jnp
from jax import lax
from jax.experimental import pallas as pl
from jax.experimental.pallas import tpu as pltpu
from jax.experimental.pallas import tpu_sc as plsc

T = 4096
D = 768
F = 2048
E = 16
K = 2

B = 256
NB = (T * K) // B + (E - 1)
NPAD = NB * B

NC = 2
NS = 16
NW = NC * NS
TOK_W = T // NW
SUB = 32

@functools.cache
def _mesh():
    return plsc.VectorSubcoreMesh(
        core_axis_name="c", subcore_axis_name="s", num_cores=NC, num_subcores=NS
    )


def _routing(topk_indices):
    flat = topk_indices.reshape(-1).astype(jnp.int32)
    onehot = (flat[:, None] == jnp.arange(E, dtype=jnp.int32)[None, :]).astype(
        jnp.int32
    )
    csum = jnp.cumsum(onehot, axis=0)
    rank = jnp.take_along_axis(csum, flat[:, None], axis=1)[:, 0] - 1
    counts = csum[-1]
    nblk = (counts + B - 1) // B
    blk_off = jnp.concatenate(
        [jnp.zeros((1,), jnp.int32), jnp.cumsum(nblk)[:-1].astype(jnp.int32)]
    )
    dst = blk_off[flat] * B + rank
    num_used = jnp.sum(nblk).astype(jnp.int32)
    bids = jnp.arange(NB, dtype=jnp.int32)
    be = jnp.sum((bids[:, None] >= blk_off[None, :]).astype(jnp.int32), axis=1) - 1
    expert_of = be[jnp.minimum(bids, num_used - 1)]
    wait_flag = jnp.concatenate(
        [jnp.ones((1,), jnp.int32),
         (expert_of[1:] != expert_of[:-1]).astype(jnp.int32)]
    )
    run_id = jnp.cumsum(wait_flag) - 1
    num_runs = run_id[-1] + 1
    expert_by_run = jnp.zeros((NB,), jnp.int32).at[run_id].set(expert_of)
    nxt = jnp.minimum(run_id + 1, num_runs - 1)
    issue_expert = expert_by_run[nxt]
    issue_flag = wait_flag * (run_id + 1 < num_runs).astype(jnp.int32)
    slot_of = run_id % 2
    issue_slot = (run_id + 1) % 2
    tab = jnp.stack(
        [expert_of, wait_flag, slot_of, issue_flag, issue_expert, issue_slot]
    )
    nu = num_used[None]
    dst2 = dst.reshape(T, K)
    return dst2[:, 0], dst2[:, 1], nu, tab


@functools.cache
def _dispatch_kernel():
    @functools.partial(
        pl.kernel,
        out_type=jax.ShapeDtypeStruct((NPAD, D), jnp.float32),
        mesh=_mesh(),
        scratch_types=[
            pltpu.VMEM((TOK_W, D), jnp.float32),
            pltpu.VMEM((TOK_W,), jnp.int32),
            pltpu.VMEM((TOK_W,), jnp.int32),
            pltpu.SemaphoreType.DMA,
            pltpu.SemaphoreType.DMA,
        ],
    )
    def _dispatch(hid_hbm, dst0_hbm, dst1_hbm, xpad_hbm, rows_v, i0_v, i1_v, s0, s1):
        wid = lax.axis_index("s") * NC + lax.axis_index("c")
        base = wid * TOK_W
        pltpu.sync_copy(hid_hbm.at[pl.ds(base, TOK_W)], rows_v)
        pltpu.sync_copy(dst0_hbm.at[pl.ds(base, TOK_W)], i0_v)
        pltpu.sync_copy(dst1_hbm.at[pl.ds(base, TOK_W)], i1_v)
        c0 = pltpu.async_copy(rows_v, xpad_hbm.at[i0_v], s0)
        c1 = pltpu.async_copy(rows_v, xpad_hbm.at[i1_v], s1)
        c0.wait()
        c1.wait()

    return _dispatch


NSUB = TOK_W // SUB


@functools.cache
def _combine_kernel():
    @functools.partial(
        pl.kernel,
        out_type=jax.ShapeDtypeStruct((T, D), jnp.float32),
        mesh=_mesh(),
        scratch_types=[
            pltpu.VMEM((2, SUB, D), jnp.float32),
            pltpu.VMEM((2, SUB, D), jnp.float32),
            pltpu.VMEM((2, SUB), jnp.int32),
            pltpu.VMEM((2, SUB), jnp.int32),
            pltpu.VMEM((2, SUB + 16), jnp.float32),
            pltpu.VMEM((2, SUB + 16), jnp.float32),
            pltpu.SemaphoreType.DMA((2,)),
            pltpu.SemaphoreType.DMA((2,)),
            pltpu.SemaphoreType.DMA((2,)),
        ],
    )
    def _combine(
        ypad_hbm, dst0_hbm, dst1_hbm, w0_hbm, w1_hbm, out_hbm,
        y0_v, y1_v, i0_v, i1_v, w0_v, w1_v, sg0, sg1, sw,
    ):
        wid = lax.axis_index("s") * NC + lax.axis_index("c")

        def issue(s, p):
            base = wid * TOK_W + s * SUB
            pltpu.sync_copy(dst0_hbm.at[pl.ds(base, SUB)], i0_v.at[p])
            pltpu.sync_copy(dst1_hbm.at[pl.ds(base, SUB)], i1_v.at[p])
            pltpu.sync_copy(w0_hbm.at[pl.ds(base, SUB)],
                            w0_v.at[p, pl.ds(0, SUB)])
            pltpu.sync_copy(w1_hbm.at[pl.ds(base, SUB)],
                            w1_v.at[p, pl.ds(0, SUB)])
            h0 = pltpu.async_copy(ypad_hbm.at[i0_v.at[p]], y0_v.at[p], sg0.at[p])
            h1 = pltpu.async_copy(ypad_hbm.at[i1_v.at[p]], y1_v.at[p], sg1.at[p])
            return h0, h1

        gh = {0: issue(0, 0)}
        wh = {}
        for s in range(NSUB):
            p = s % 2
            if s + 1 < NSUB:
                if s - 1 in wh:
                    wh.pop(s - 1).wait()
                gh[s + 1] = issue(s + 1, (s + 1) % 2)
            h0, h1 = gh.pop(s)
            h0.wait()
            h1.wait()

            def row_body(r, carry):
                a = w0_v[p, pl.ds(r, 16)][0]
                b = w1_v[p, pl.ds(r, 16)][0]
                for cb in range(D // 16):
                    sl = pl.ds(cb * 16, 16)
                    y0_v[p, r, sl] = a * y0_v[p, r, sl] + b * y1_v[p, r, sl]
                return carry

            lax.fori_loop(0, SUB, row_body, 0)
            base = wid * TOK_W + s * SUB
            wh[s] = pltpu.async_copy(
                y0_v.at[p], out_hbm.at[pl.ds(base, SUB)], sw.at[p]
            )
        for s in sorted(wh):
            wh.pop(s).wait()

    return _combine


def _ffn_body(
    nu_ref, tab_ref, x_ref, wg_hbm, wu_hbm, wd_hbm, y_ref,
    wg_v, wu_v, wd_v, sg, su, sd,
):
    b = pl.program_id(0)

    def start_fetch(e, slot):
        pltpu.make_async_copy(wg_hbm.at[e], wg_v.at[slot], sg.at[slot]).start()
        pltpu.make_async_copy(wu_hbm.at[e], wu_v.at[slot], su.at[slot]).start()
        pltpu.make_async_copy(wd_hbm.at[e], wd_v.at[slot], sd.at[slot]).start()

    def wait_fetch(slot):
        pltpu.make_async_copy(wg_hbm.at[0], wg_v.at[slot], sg.at[slot]).wait()
        pltpu.make_async_copy(wu_hbm.at[0], wu_v.at[slot], su.at[slot]).wait()
        pltpu.make_async_copy(wd_hbm.at[0], wd_v.at[slot], sd.at[slot]).wait()

    slot = tab_ref[2, b]

    @pl.when(b == 0)
    def _():
        start_fetch(tab_ref[0, 0], slot)

    @pl.when(tab_ref[3, b] == 1)
    def _():
        start_fetch(tab_ref[4, b], tab_ref[5, b])

    @pl.when(tab_ref[1, b] == 1)
    def _():
        wait_fetch(slot)

    @pl.when(b < nu_ref[0])
    def _():
        x = x_ref[...]
        g = jnp.dot(x, wg_v[slot], preferred_element_type=jnp.float32)
        u = jnp.dot(x, wu_v[slot], preferred_element_type=jnp.float32)
        h = g * jax.nn.sigmoid(g) * u
        y_ref[...] = jnp.dot(h, wd_v[slot], preferred_element_type=jnp.float32)


def _ffn(nu, tab, x_pad, W_gate, W_up, W_down):
    grid_spec = pltpu.PrefetchScalarGridSpec(
        num_scalar_prefetch=2,
        grid=(NB,),
        in_specs=[
            pl.BlockSpec((B, D), lambda b, nu_, m: (jnp.minimum(b, nu_[0] - 1), 0)),
            pl.BlockSpec(memory_space=pltpu.HBM),
            pl.BlockSpec(memory_space=pltpu.HBM),
            pl.BlockSpec(memory_space=pltpu.HBM),
        ],
        out_specs=pl.BlockSpec(
            (B, D), lambda b, nu_, m: (jnp.minimum(b, nu_[0] - 1), 0)
        ),
        scratch_shapes=[
            pltpu.VMEM((2, D, F), jnp.float32),
            pltpu.VMEM((2, D, F), jnp.float32),
            pltpu.VMEM((2, F, D), jnp.float32),
            pltpu.SemaphoreType.DMA((2,)),
            pltpu.SemaphoreType.DMA((2,)),
            pltpu.SemaphoreType.DMA((2,)),
        ],
    )
    return pl.pallas_call(
        _ffn_body,
        grid_spec=grid_spec,
        out_shape=jax.ShapeDtypeStruct((NPAD, D), jnp.float32),
        compiler_params=pltpu.CompilerParams(
            dimension_semantics=("arbitrary",),
            vmem_limit_bytes=100 * 1024 * 1024,
        ),
    )(nu, tab, x_pad, W_gate, W_up, W_down)


def kernel(hidden_states, topk_indices, topk_weights, W_gate, W_up, W_down):
    dst0, dst1, nu, tab = _routing(topk_indices)
    x_pad = _dispatch_kernel()(hidden_states, dst0, dst1)
    y_pad = _ffn(nu, tab, x_pad, W_gate, W_up, W_down)
    w = topk_weights.astype(jnp.float32)
    out = _combine_kernel()(y_pad, dst0, dst1, w[:, 0], w[:, 1])
    return out


# speedup vs baseline: 1.1790x; 1.1790x over previous
import functools

import jax
import jax.numpy as jnp
from jax import lax
from jax.experimental import pallas as pl
from jax.experimental.pallas import tpu as pltpu
from jax.experimental.pallas import tpu_sc as plsc

T = 4096
D = 768
F = 2048
E = 16
K = 2

B = 256
NB = (T * K) // B + (E - 1)
NPAD = NB * B

NC = 2
NS = 16
NW = NC * NS
TOK_W = T // NW
SUB = 64

@functools.cache
def _mesh():
    return plsc.VectorSubcoreMesh(
        core_axis_name="c", subcore_axis_name="s", num_cores=NC, num_subcores=NS
    )


def _routing(topk_indices):
    flat = topk_indices.reshape(-1).astype(jnp.int32)
    oh = (jnp.arange(E, dtype=jnp.int32)[:, None] == flat[None, :]).astype(
        jnp.int32
    )
    csum = jnp.cumsum(oh, axis=1)
    counts = csum[:, -1]
    nblk = (counts + B - 1) // B
    blk_off = (jnp.cumsum(nblk) - nblk).astype(jnp.int32)
    dst = jnp.sum(oh * (blk_off[:, None] * B + csum - 1), axis=0)
    num_used = jnp.sum(nblk).astype(jnp.int32)
    bids = jnp.arange(NB, dtype=jnp.int32)
    be = jnp.sum((bids[:, None] >= blk_off[None, :]).astype(jnp.int32), axis=1) - 1
    expert_of = be[jnp.minimum(bids, num_used - 1)]
    wait_flag = jnp.concatenate(
        [jnp.ones((1,), jnp.int32),
         (expert_of[1:] != expert_of[:-1]).astype(jnp.int32)]
    )
    run_id = jnp.cumsum(wait_flag) - 1
    num_runs = run_id[-1] + 1
    expert_by_run = jnp.zeros((NB,), jnp.int32).at[run_id].set(expert_of)
    nxt = jnp.minimum(run_id + 1, num_runs - 1)
    issue_expert = expert_by_run[nxt]
    issue_flag = wait_flag * (run_id + 1 < num_runs).astype(jnp.int32)
    slot_of = run_id % 2
    issue_slot = (run_id + 1) % 2
    tab = jnp.stack(
        [expert_of, wait_flag, slot_of, issue_flag, issue_expert, issue_slot]
    )
    nu = num_used[None]
    dst2 = dst.reshape(T, K)
    return dst2[:, 0], dst2[:, 1], nu, tab


@functools.cache
def _dispatch_kernel():
    @functools.partial(
        pl.kernel,
        out_type=jax.ShapeDtypeStruct((NPAD, D), jnp.float32),
        mesh=_mesh(),
        scratch_types=[
            pltpu.VMEM((TOK_W, D), jnp.float32),
            pltpu.VMEM((TOK_W,), jnp.int32),
            pltpu.VMEM((TOK_W,), jnp.int32),
            pltpu.SemaphoreType.DMA,
            pltpu.SemaphoreType.DMA,
        ],
    )
    def _dispatch(hid_hbm, dst0_hbm, dst1_hbm, xpad_hbm, rows_v, i0_v, i1_v, s0, s1):
        wid = lax.axis_index("s") * NC + lax.axis_index("c")
        base = wid * TOK_W
        pltpu.sync_copy(hid_hbm.at[pl.ds(base, TOK_W)], rows_v)
        pltpu.sync_copy(dst0_hbm.at[pl.ds(base, TOK_W)], i0_v)
        pltpu.sync_copy(dst1_hbm.at[pl.ds(base, TOK_W)], i1_v)
        c0 = pltpu.async_copy(rows_v, xpad_hbm.at[i0_v], s0)
        c1 = pltpu.async_copy(rows_v, xpad_hbm.at[i1_v], s1)
        c0.wait()
        c1.wait()

    return _dispatch


@functools.cache
def _combine_kernel():
    @functools.partial(
        pl.kernel,
        out_type=jax.ShapeDtypeStruct((T, D), jnp.float32),
        mesh=_mesh(),
        scratch_types=[
            pltpu.VMEM((SUB, D), jnp.float32),
            pltpu.VMEM((SUB, D), jnp.float32),
            pltpu.VMEM((SUB,), jnp.int32),
            pltpu.VMEM((SUB,), jnp.int32),
            pltpu.VMEM((SUB + 16,), jnp.float32),
            pltpu.VMEM((SUB + 16,), jnp.float32),
            pltpu.SemaphoreType.DMA,
            pltpu.SemaphoreType.DMA,
        ],
    )
    def _combine(
        ypad_hbm, dst0_hbm, dst1_hbm, w0_hbm, w1_hbm, out_hbm,
        y0_v, y1_v, i0_v, i1_v, w0_v, w1_v, s0, s1,
    ):
        wid = lax.axis_index("s") * NC + lax.axis_index("c")
        for sub in range(TOK_W // SUB):
            base = wid * TOK_W + sub * SUB
            pltpu.sync_copy(dst0_hbm.at[pl.ds(base, SUB)], i0_v)
            pltpu.sync_copy(dst1_hbm.at[pl.ds(base, SUB)], i1_v)
            pltpu.sync_copy(w0_hbm.at[pl.ds(base, SUB)], w0_v.at[pl.ds(0, SUB)])
            pltpu.sync_copy(w1_hbm.at[pl.ds(base, SUB)], w1_v.at[pl.ds(0, SUB)])
            pltpu.async_copy(ypad_hbm.at[i0_v], y0_v, s0).wait()
            pltpu.async_copy(ypad_hbm.at[i1_v], y1_v, s1).wait()

            def row_body(r, carry):
                a = w0_v[pl.ds(r, 16)][0]
                b = w1_v[pl.ds(r, 16)][0]
                for cb in range(D // 16):
                    sl = pl.ds(cb * 16, 16)
                    y0_v[r, sl] = a * y0_v[r, sl] + b * y1_v[r, sl]
                return carry

            lax.fori_loop(0, SUB, row_body, 0)
            pltpu.sync_copy(y0_v, out_hbm.at[pl.ds(base, SUB)])

    return _combine


def _ffn_body(
    nu_ref, tab_ref, x_ref, wg_hbm, wu_hbm, wd_hbm, y_ref,
    wg_v, wu_v, wd_v, sg, su, sd,
):
    b = pl.program_id(0)

    def start_fetch(e, slot):
        pltpu.make_async_copy(wg_hbm.at[e], wg_v.at[slot], sg.at[slot]).start()
        pltpu.make_async_copy(wu_hbm.at[e], wu_v.at[slot], su.at[slot]).start()
        pltpu.make_async_copy(wd_hbm.at[e], wd_v.at[slot], sd.at[slot]).start()

    def wait_fetch(slot):
        pltpu.make_async_copy(wg_hbm.at[0], wg_v.at[slot], sg.at[slot]).wait()
        pltpu.make_async_copy(wu_hbm.at[0], wu_v.at[slot], su.at[slot]).wait()
        pltpu.make_async_copy(wd_hbm.at[0], wd_v.at[slot], sd.at[slot]).wait()

    slot = tab_ref[2, b]

    @pl.when(b == 0)
    def _():
        start_fetch(tab_ref[0, 0], slot)

    @pl.when(tab_ref[3, b] == 1)
    def _():
        start_fetch(tab_ref[4, b], tab_ref[5, b])

    @pl.when(tab_ref[1, b] == 1)
    def _():
        wait_fetch(slot)

    @pl.when(b < nu_ref[0])
    def _():
        x = x_ref[...]
        g = jnp.dot(x, wg_v[slot], preferred_element_type=jnp.float32)
        u = jnp.dot(x, wu_v[slot], preferred_element_type=jnp.float32)
        h = g * jax.nn.sigmoid(g) * u
        y_ref[...] = jnp.dot(h, wd_v[slot], preferred_element_type=jnp.float32)


def _ffn(nu, tab, x_pad, W_gate, W_up, W_down):
    grid_spec = pltpu.PrefetchScalarGridSpec(
        num_scalar_prefetch=2,
        grid=(NB,),
        in_specs=[
            pl.BlockSpec((B, D), lambda b, nu_, m: (jnp.minimum(b, nu_[0] - 1), 0)),
            pl.BlockSpec(memory_space=pltpu.HBM),
            pl.BlockSpec(memory_space=pltpu.HBM),
            pl.BlockSpec(memory_space=pltpu.HBM),
        ],
        out_specs=pl.BlockSpec(
            (B, D), lambda b, nu_, m: (jnp.minimum(b, nu_[0] - 1), 0)
        ),
        scratch_shapes=[
            pltpu.VMEM((2, D, F), jnp.float32),
            pltpu.VMEM((2, D, F), jnp.float32),
            pltpu.VMEM((2, F, D), jnp.float32),
            pltpu.SemaphoreType.DMA((2,)),
            pltpu.SemaphoreType.DMA((2,)),
            pltpu.SemaphoreType.DMA((2,)),
        ],
    )
    return pl.pallas_call(
        _ffn_body,
        grid_spec=grid_spec,
        out_shape=jax.ShapeDtypeStruct((NPAD, D), jnp.float32),
        compiler_params=pltpu.CompilerParams(
            dimension_semantics=("arbitrary",),
            vmem_limit_bytes=100 * 1024 * 1024,
        ),
    )(nu, tab, x_pad, W_gate, W_up, W_down)


def kernel(hidden_states, topk_indices, topk_weights, W_gate, W_up, W_down):
    dst0, dst1, nu, tab = _routing(topk_indices)
    x_pad = _dispatch_kernel()(hidden_states, dst0, dst1)
    y_pad = _ffn(nu, tab, x_pad, W_gate, W_up, W_down)
    w = topk_weights.astype(jnp.float32)
    out = _combine_kernel()(y_pad, dst0, dst1, w[:, 0], w[:, 1])
    return out
